# Initial kernel scaffold; baseline (speedup 1.0000x reference)
import functools

import jax
import jax.numpy as jnp
from jax import lax
from jax.experimental import pallas as pl
from jax.experimental.pallas import tpu as pltpu
from jax.experimental.pallas import tpu_sc as plsc


def _build_gather(N, V, D, nc, ns):
    NW = nc * ns
    n_per_w = N // NW
    C = 1600
    assert n_per_w % C == 0
    nchunks = n_per_w // C

    mesh = plsc.VectorSubcoreMesh(core_axis_name="c", subcore_axis_name="s")

    @functools.partial(
        pl.kernel,
        out_type=jax.ShapeDtypeStruct((N, D), jnp.float32),
        mesh=mesh,
        scratch_types=[
            pltpu.VMEM((n_per_w,), jnp.int32),
            pltpu.VMEM((2, C, D), jnp.float32),
            pltpu.SemaphoreType.DMA,
            pltpu.SemaphoreType.DMA,
            pltpu.SemaphoreType.DMA,
            pltpu.SemaphoreType.DMA,
        ],
    )
    def gather_kernel(idx_hbm, table_hbm, out_hbm, idx_v, rows_v, g0, g1, w0, w1):
        wid = lax.axis_index("s") * nc + lax.axis_index("c")
        base = wid * n_per_w
        pltpu.sync_copy(idx_hbm.at[pl.ds(base, n_per_w)], idx_v)

        gsem = [g0, g1]
        wsem = [w0, w1]

        def gather(i):
            b = i % 2
            return pltpu.async_copy(
                table_hbm.at[idx_v.at[pl.ds(i * C, C)]], rows_v.at[b], gsem[b]
            )

        def writeback(i):
            b = i % 2
            return pltpu.async_copy(
                rows_v.at[b], out_hbm.at[pl.ds(base + i * C, C)], wsem[b]
            )

        gh = gather(0)
        wh = [None, None]
        for i in range(nchunks):
            gh.wait()
            if i + 1 < nchunks:
                nb = (i + 1) % 2
                if wh[nb] is not None:
                    wh[nb].wait()
                    wh[nb] = None
                gh = gather(i + 1)
            wh[i % 2] = writeback(i)
        for h in wh:
            if h is not None:
                h.wait()

    return gather_kernel


def kernel(indices, mask, table):
    B, L, NV = indices.shape
    V, D = table.shape
    N = B * L * NV
    info = plsc.get_sparse_core_info()
    gather_fn = _build_gather(N, V, D, info.num_cores, info.num_subcores)
    out = gather_fn(indices.reshape(N), table)
    return out.reshape(B, L, NV * D), mask


# speedup vs baseline: 2.2102x; 2.2102x over previous
import functools

import jax
import jax.numpy as jnp
from jax import lax
from jax.experimental import pallas as pl
from jax.experimental.pallas import tpu as pltpu
from jax.experimental.pallas import tpu_sc as plsc


def _build_gather(N, V, D, nc, ns):
    NW = nc * ns
    n_per_w = N // NW
    C = 1600
    assert n_per_w % C == 0
    nchunks = n_per_w // C

    mesh = plsc.VectorSubcoreMesh(core_axis_name="c", subcore_axis_name="s")

    @functools.partial(
        pl.kernel,
        out_type=jax.ShapeDtypeStruct((N, D), jnp.float32),
        mesh=mesh,
        compiler_params=pltpu.CompilerParams(use_tc_tiling_on_sc=False),
        scratch_types=[
            pltpu.VMEM((n_per_w,), jnp.int32),
            pltpu.VMEM((2, C, D), jnp.float32),
            pltpu.SemaphoreType.DMA,
            pltpu.SemaphoreType.DMA,
            pltpu.SemaphoreType.DMA,
            pltpu.SemaphoreType.DMA,
        ],
    )
    def gather_kernel(idx_hbm, table_hbm, out_hbm, idx_v, rows_v, g0, g1, w0, w1):
        wid = lax.axis_index("s") * nc + lax.axis_index("c")
        base = wid * n_per_w
        pltpu.sync_copy(idx_hbm.at[pl.ds(base, n_per_w)], idx_v)

        gsem = [g0, g1]
        wsem = [w0, w1]

        def gather(i):
            b = i % 2
            return pltpu.async_copy(
                table_hbm.at[idx_v.at[pl.ds(i * C, C)]], rows_v.at[b], gsem[b]
            )

        def writeback(i):
            b = i % 2
            return pltpu.async_copy(
                rows_v.at[b], out_hbm.at[pl.ds(base + i * C, C)], wsem[b]
            )

        gh = gather(0)
        wh = [None, None]
        for i in range(nchunks):
            gh.wait()
            if i + 1 < nchunks:
                nb = (i + 1) % 2
                if wh[nb] is not None:
                    wh[nb].wait()
                    wh[nb] = None
                gh = gather(i + 1)
            wh[i % 2] = writeback(i)
        for h in wh:
            if h is not None:
                h.wait()

    return gather_kernel


def kernel(indices, mask, table):
    B, L, NV = indices.shape
    V, D = table.shape
    N = B * L * NV
    info = plsc.get_sparse_core_info()
    gather_fn = _build_gather(N, V, D, info.num_cores, info.num_subcores)
    out = gather_fn(indices.reshape(N), table)
    return out.reshape(B, L, NV * D), mask


# speedup vs baseline: 2.2210x; 1.0049x over previous
import functools

import jax
import jax.numpy as jnp
from jax import lax
from jax.experimental import pallas as pl
from jax.experimental.pallas import tpu as pltpu
from jax.experimental.pallas import tpu_sc as plsc


def _build_gather(N, V, D, nc, ns):
    NW = nc * ns
    n_per_w = N // NW
    C = 800
    NBUF = 4
    K = 2
    assert n_per_w % C == 0
    nchunks = n_per_w // C

    mesh = plsc.VectorSubcoreMesh(core_axis_name="c", subcore_axis_name="s")

    @functools.partial(
        pl.kernel,
        out_type=jax.ShapeDtypeStruct((N, D), jnp.float32),
        mesh=mesh,
        compiler_params=pltpu.CompilerParams(use_tc_tiling_on_sc=False),
        scratch_types=[
            pltpu.VMEM((n_per_w,), jnp.int32),
            pltpu.VMEM((NBUF, C, D), jnp.float32),
            [pltpu.SemaphoreType.DMA] * NBUF,
            [pltpu.SemaphoreType.DMA] * NBUF,
        ],
    )
    def gather_kernel(idx_hbm, table_hbm, out_hbm, idx_v, rows_v, gsem, wsem):
        wid = lax.axis_index("s") * nc + lax.axis_index("c")
        base = wid * n_per_w
        pltpu.sync_copy(idx_hbm.at[pl.ds(base, n_per_w)], idx_v)

        def gather(i):
            b = i % NBUF
            return pltpu.async_copy(
                table_hbm.at[idx_v.at[pl.ds(i * C, C)]], rows_v.at[b], gsem[b]
            )

        def writeback(i):
            b = i % NBUF
            return pltpu.async_copy(
                rows_v.at[b], out_hbm.at[pl.ds(base + i * C, C)], wsem[b]
            )

        gh = [None] * nchunks
        wh = [None] * nchunks
        for i in range(nchunks + K):
            if i < nchunks:
                if i >= NBUF:
                    wh[i - NBUF].wait()
                gh[i] = gather(i)
            j = i - K
            if 0 <= j:
                gh[j].wait()
                wh[j] = writeback(j)
        for j in range(nchunks - NBUF, nchunks):
            wh[j].wait()

    return gather_kernel


def kernel(indices, mask, table):
    B, L, NV = indices.shape
    V, D = table.shape
    N = B * L * NV
    info = plsc.get_sparse_core_info()
    gather_fn = _build_gather(N, V, D, info.num_cores, info.num_subcores)
    out = gather_fn(indices.reshape(N), table)
    return out.reshape(B, L, NV * D), mask
